# baseline (speedup 1.0000x reference)
import functools

import jax
import jax.numpy as jnp
from jax import lax
from jax.experimental import pallas as pl

N = 10000
E = 320000
D_IN = 128
D1 = 128
D2 = 32
ENC = 64

TN_B = 1000
FC_COLS = 6400
TE_F = 8000


def _dot_t(a, b):
    return lax.dot_general(a, b, (((1,), (1,)), ((), ())),
                           preferred_element_type=jnp.float32)


def _conv_dense_body(s1_ref, cnt_ref, x_ref, w1r_ref, w1t_ref, b1_ref,
                     w2r_ref, w2t_ref, b2_ref, hp_ref, hrb_ref):
    cnt = jnp.maximum(cnt_ref[...], 1.0)
    agg = s1_ref[...] / cnt
    h = jnp.maximum(
        _dot_t(agg, w1r_ref[...]) + _dot_t(x_ref[...], w1t_ref[...]) + b1_ref[...],
        0.0)
    hp_ref[...] = _dot_t(h, w2r_ref[...])
    hrb_ref[...] = _dot_t(h, w2t_ref[...]) + b2_ref[...]


def _conv_dense(s1, cnt, x, W1_rel, W1_root, b1, W2_rel, W2_root, b2):
    grid = N // TN_B
    row = lambda i: (i, 0)
    whole = lambda i: (0, 0)
    return pl.pallas_call(
        _conv_dense_body,
        grid=(grid,),
        in_specs=[
            pl.BlockSpec((TN_B, D1), row),
            pl.BlockSpec((TN_B, 1), row),
            pl.BlockSpec((TN_B, D_IN), row),
            pl.BlockSpec((D1, D_IN), whole),
            pl.BlockSpec((D1, D_IN), whole),
            pl.BlockSpec((1, D1), whole),
            pl.BlockSpec((D2, D1), whole),
            pl.BlockSpec((D2, D1), whole),
            pl.BlockSpec((1, D2), whole),
        ],
        out_specs=[
            pl.BlockSpec((TN_B, D2), row),
            pl.BlockSpec((TN_B, D2), row),
        ],
        out_shape=[
            jax.ShapeDtypeStruct((N, D2), jnp.float32),
            jax.ShapeDtypeStruct((N, D2), jnp.float32),
        ],
    )(s1, cnt, x, W1_rel, W1_root, b1, W2_rel, W2_root, b2)


def _combine2_body(s2_ref, cnt_ref, hrb_ref, zn_ref):
    cnt = jnp.maximum(cnt_ref[...], 1.0)
    zn_ref[...] = jnp.maximum(s2_ref[...] / cnt + hrb_ref[...], 0.0)


def _combine2(s2, cnt, hrb):
    grid = N // TN_B
    row = lambda i: (i, 0)
    return pl.pallas_call(
        _combine2_body,
        grid=(grid,),
        in_specs=[
            pl.BlockSpec((TN_B, D2), row),
            pl.BlockSpec((TN_B, 1), row),
            pl.BlockSpec((TN_B, D2), row),
        ],
        out_specs=pl.BlockSpec((TN_B, D2), row),
        out_shape=jax.ShapeDtypeStruct((N, D2), jnp.float32),
    )(s2, cnt, hrb)


def _fc_body(z_ref, fcw_ref, femb_ref, emb_ref, fcb_ref, out_ref):
    i = pl.program_id(0)

    @pl.when(i == 0)
    def _init():
        out_ref[...] = fcb_ref[...] + _dot_t(emb_ref[...], femb_ref[...])

    out_ref[...] += _dot_t(z_ref[...], fcw_ref[...])


def _fc_matvec(z, fc_W, femb, embcat, fc_b):
    grid = N * D2 // FC_COLS
    return pl.pallas_call(
        _fc_body,
        grid=(grid,),
        in_specs=[
            pl.BlockSpec((1, FC_COLS), lambda i: (0, i)),
            pl.BlockSpec((ENC, FC_COLS), lambda i: (0, i)),
            pl.BlockSpec((ENC, 200), lambda i: (0, 0)),
            pl.BlockSpec((1, 200), lambda i: (0, 0)),
            pl.BlockSpec((1, ENC), lambda i: (0, 0)),
        ],
        out_specs=pl.BlockSpec((1, ENC), lambda i: (0, 0)),
        out_shape=jax.ShapeDtypeStruct((1, ENC), jnp.float32),
    )(z, fc_W, femb, embcat, fc_b)


def _fc2_body(fc2w_ref, s_ref, emb_ref, fc2b_ref, out_ref):
    w = fc2w_ref[...]
    t = (lax.dot_general(w[:, :ENC], s_ref[...], (((1,), (1,)), ((), ())),
                         preferred_element_type=jnp.float32)
         + lax.dot_general(w[:, ENC:], emb_ref[...], (((1,), (1,)), ((), ())),
                           preferred_element_type=jnp.float32)
         + fc2b_ref[...])
    out_ref[...] = jnp.maximum(t, 0.0)


def _fc2_matvec(fc2_W, s, embcat, fc2_b):
    grid = N * D2 // FC_COLS
    return pl.pallas_call(
        _fc2_body,
        grid=(grid,),
        in_specs=[
            pl.BlockSpec((FC_COLS, ENC + 200), lambda i: (i, 0)),
            pl.BlockSpec((1, ENC), lambda i: (0, 0)),
            pl.BlockSpec((1, 200), lambda i: (0, 0)),
            pl.BlockSpec((FC_COLS, 1), lambda i: (i, 0)),
        ],
        out_specs=pl.BlockSpec((FC_COLS, 1), lambda i: (i, 0)),
        out_shape=jax.ShapeDtypeStruct((N * D2, 1), jnp.float32),
    )(fc2_W, s, embcat, fc2_b)


def _decoder_body(gs_ref, gd_ref, w1_ref, b1_ref, w2_ref, b2_ref, out_ref):
    w1 = w1_ref[...]
    e = jnp.maximum(
        _dot_t(gs_ref[...], w1[:, :D2]) + _dot_t(gd_ref[...], w1[:, D2:])
        + b1_ref[...], 0.0)
    t = lax.dot_general(e, w2_ref[...], (((1,), (0,)), ((), ())),
                        preferred_element_type=jnp.float32) + b2_ref[...]
    out_ref[...] = jax.nn.sigmoid(t)


def _decoder(gs, gd, dec_W1, dec_b1, dec_w2c, dec_b2):
    grid = E // TE_F
    return pl.pallas_call(
        _decoder_body,
        grid=(grid,),
        in_specs=[
            pl.BlockSpec((TE_F, D2), lambda i: (i, 0)),
            pl.BlockSpec((TE_F, D2), lambda i: (i, 0)),
            pl.BlockSpec((D1, 2 * D2), lambda i: (0, 0)),
            pl.BlockSpec((1, D1), lambda i: (0, 0)),
            pl.BlockSpec((D1, 1), lambda i: (0, 0)),
            pl.BlockSpec((1, 1), lambda i: (0, 0)),
        ],
        out_specs=pl.BlockSpec((TE_F, 1), lambda i: (i, 0)),
        out_shape=jax.ShapeDtypeStruct((E, 1), jnp.float32),
    )(gs, gd, dec_W1, dec_b1, dec_w2c, dec_b2)


def kernel(x, edge_index, edge_attr, hour, week, W1_rel, b1_rel, W1_root,
           b1_root, W2_rel, b2_rel, W2_root, b2_root, hour_table, week_table,
           fc_W, fc_b, fc2_W, fc2_b, dec_W1, dec_b1, dec_W2, dec_b2):
    src = edge_index[0]
    dst = edge_index[1]
    ew = edge_attr

    cnt = jax.ops.segment_sum(jnp.ones((E,), jnp.float32), dst, num_segments=N)
    s1 = jax.ops.segment_sum(x[src] * ew[:, None], dst, num_segments=N)
    cnt2d = cnt.reshape(N, 1)

    b1 = (b1_rel + b1_root).reshape(1, D1)
    b2 = (b2_rel + b2_root).reshape(1, D2)
    hp, hrb = _conv_dense(s1, cnt2d, x, W1_rel, W1_root, b1,
                          W2_rel, W2_root, b2)

    s2 = jax.ops.segment_sum(hp[src] * ew[:, None], dst, num_segments=N)

    zn = _combine2(s2, cnt2d, hrb)

    emb_h = hour_table[hour]
    emb_w = week_table[week]
    embcat = jnp.concatenate([emb_h, emb_w], axis=-1)

    z = zn.reshape(1, N * D2)
    femb = fc_W[:, N * D2:]
    s_enc = _fc_matvec(z, fc_W, femb, embcat, fc_b.reshape(1, ENC))

    z2 = _fc2_matvec(fc2_W, s_enc, embcat, fc2_b.reshape(N * D2, 1))
    zn2 = z2.reshape(N, D2)

    gs = zn2[src]
    gd = zn2[dst]

    pred = _decoder(gs, gd, dec_W1, dec_b1.reshape(1, D1),
                    dec_W2.reshape(D1, 1), dec_b2.reshape(1, 1))
    return pred.reshape(E)
